# baseline (speedup 1.0000x reference)
import functools

import jax
import jax.numpy as jnp
import numpy as np
from jax import lax
from jax.experimental import pallas as pl
from jax.experimental.pallas import tpu as pltpu
from jax.experimental.pallas import tpu_sc as plsc

NBINS = 256
LANES = 16
NCORES = 2
NSUB = 16
NWORKERS = NCORES * NSUB
SUBHIST = NBINS * LANES
CHUNK = 16384
UNROLL = 16


def _hist_body(nchunk, x_hbm, out_hbm, buf0, buf1, hist_v, sem0, sem1):
    cid = lax.axis_index("c")
    sid = lax.axis_index("s")
    wid = sid * NCORES + cid
    per_w = nchunk * CHUNK
    base = wid * per_w

    zeros16 = jnp.zeros((LANES,), jnp.float32)
    ones16 = jnp.ones((LANES,), jnp.float32)
    lane = lax.iota(jnp.int32, LANES)
    lane_magic = lane.astype(jnp.float32) + jnp.float32(8388608.0)

    def zero_body(i, carry):
        for k in range(16):
            hist_v[pl.ds(i * (16 * LANES) + k * LANES, LANES)] = zeros16
        return carry

    lax.fori_loop(0, SUBHIST // (16 * LANES), zero_body, 0)

    def start(g, buf, sem):
        pltpu.async_copy(x_hbm.at[pl.ds(base + g * CHUNK, CHUNK)], buf, sem)

    def wait(buf, sem):
        pltpu.make_async_copy(x_hbm.at[pl.ds(base, CHUNK)], buf, sem).wait()

    def process(buf):
        def vec_body(i, c2):
            b0 = i * (LANES * UNROLL)
            vs = [buf[pl.ds(b0 + k * LANES, LANES)] for k in range(UNROLL)]
            fs = [v * jnp.float32(16.0) + lane_magic for v in vs]
            idxs = [plsc.bitcast(f, jnp.int32) & jnp.int32(0xFFF) for f in fs]
            for idx in idxs:
                plsc.addupdate_scatter(hist_v, [idx], ones16)
            return c2

        lax.fori_loop(0, CHUNK // (LANES * UNROLL), vec_body, 0)

    start(0, buf0, sem0)

    def pair_body(p, carry):
        g0 = 2 * p
        wait(buf0, sem0)
        start(g0 + 1, buf1, sem1)
        process(buf0)
        wait(buf1, sem1)

        @pl.when(g0 + 2 < nchunk)
        def _():
            start(g0 + 2, buf0, sem0)

        process(buf1)
        return carry

    lax.fori_loop(0, nchunk // 2, pair_body, 0)
    pltpu.sync_copy(hist_v, out_hbm.at[wid])


def _make_hist(n):
    per_w = n // NWORKERS
    assert per_w * NWORKERS == n and per_w % (2 * CHUNK) == 0
    nchunk = per_w // CHUNK
    return pl.kernel(
        functools.partial(_hist_body, nchunk),
        out_type=jax.ShapeDtypeStruct((NWORKERS, SUBHIST), jnp.float32),
        mesh=plsc.VectorSubcoreMesh(
            core_axis_name="c", subcore_axis_name="s",
            num_cores=NCORES, num_subcores=NSUB,
        ),
        scratch_types=[
            pltpu.VMEM((CHUNK,), jnp.float32),
            pltpu.VMEM((CHUNK,), jnp.float32),
            pltpu.VMEM((SUBHIST,), jnp.float32),
            pltpu.SemaphoreType.DMA,
            pltpu.SemaphoreType.DMA,
        ],
        compiler_params=pltpu.CompilerParams(needs_layout_passes=False),
    )


def _otsu_body(hists_ref, out_ref):
    a = hists_ref[...]
    c = jnp.sum(a, axis=0, keepdims=True)
    jv = (lax.broadcasted_iota(jnp.int32, (1, SUBHIST), 1) >> 4).astype(
        jnp.float32)
    jj = lax.broadcasted_iota(jnp.int32, (SUBHIST, 1), 0) >> 4
    ii = lax.broadcasted_iota(jnp.int32, (1, NBINS), 1)
    k1 = (jj <= ii).astype(jnp.float32)

    w1 = lax.dot_general(
        c, k1, (((1,), (0,)), ((), ())),
        precision=lax.Precision.HIGHEST, preferred_element_type=jnp.float32,
    )
    cs1 = lax.dot_general(
        c * jv, k1, (((1,), (0,)), ((), ())),
        precision=lax.Precision.HIGHEST, preferred_element_type=jnp.float32,
    )

    n_tot = jnp.max(w1)
    s_tot = jnp.max(cs1)
    w2 = n_tot - w1
    valid = jnp.logical_and(w1 > 0.0, w2 > 0.0)
    var12 = w1 * w2 * (cs1 / w1 - (s_tot - cs1) / w2) ** 2
    var12 = jnp.where(valid, var12, -jnp.inf)
    m = jnp.max(var12)
    centers = lax.broadcasted_iota(jnp.int32, (1, NBINS), 1).astype(jnp.float32)
    cand = jnp.where(var12 == m, centers, jnp.float32(NBINS))
    out_ref[...] = jnp.reshape(jnp.min(cand), (1, 1))


_otsu = pl.pallas_call(
    _otsu_body,
    out_shape=jax.ShapeDtypeStruct((1, 1), jnp.float32),
)


@jax.jit
def kernel(x):
    hists = _make_hist(x.size)(x)
    thr = _otsu(hists)
    return thr[0, 0]


# speedup vs baseline: 1.0734x; 1.0734x over previous
import functools

import jax
import jax.numpy as jnp
import numpy as np
from jax import lax
from jax.experimental import pallas as pl
from jax.experimental.pallas import tpu as pltpu
from jax.experimental.pallas import tpu_sc as plsc

NBINS = 256
LANES = 16
NCORES = 2
NSUB = 16
NWORKERS = NCORES * NSUB
SUBHIST = NBINS * LANES
CHUNK = 32768
UNROLL = 16


def _hist_body(nchunk, x_hbm, out_hbm, buf0, buf1, hist_v, sem0, sem1):
    cid = lax.axis_index("c")
    sid = lax.axis_index("s")
    wid = sid * NCORES + cid
    per_w = nchunk * CHUNK
    base = wid * per_w

    zeros16 = jnp.zeros((LANES,), jnp.float32)
    ones16 = jnp.ones((LANES,), jnp.float32)
    lane = lax.iota(jnp.int32, LANES)
    lane_magic = lane.astype(jnp.float32) + jnp.float32(8388608.0)

    def zero_body(i, carry):
        for k in range(16):
            hist_v[pl.ds(i * (16 * LANES) + k * LANES, LANES)] = zeros16
        return carry

    lax.fori_loop(0, SUBHIST // (16 * LANES), zero_body, 0)

    def start(g, buf, sem):
        pltpu.async_copy(x_hbm.at[pl.ds(base + g * CHUNK, CHUNK)], buf, sem)

    def wait(buf, sem):
        pltpu.make_async_copy(x_hbm.at[pl.ds(base, CHUNK)], buf, sem).wait()

    def process(buf):
        def vec_body(i, c2):
            b0 = i * (LANES * UNROLL)
            vs = [buf[pl.ds(b0 + k * LANES, LANES)] for k in range(UNROLL)]
            fs = [v * jnp.float32(16.0) + lane_magic for v in vs]
            idxs = [plsc.bitcast(f, jnp.int32) & jnp.int32(0xFFF) for f in fs]
            for idx in idxs:
                plsc.addupdate_scatter(hist_v, [idx], ones16)
            return c2

        lax.fori_loop(0, CHUNK // (LANES * UNROLL), vec_body, 0)

    start(0, buf0, sem0)

    def pair_body(p, carry):
        g0 = 2 * p
        wait(buf0, sem0)
        start(g0 + 1, buf1, sem1)
        process(buf0)
        wait(buf1, sem1)

        @pl.when(g0 + 2 < nchunk)
        def _():
            start(g0 + 2, buf0, sem0)

        process(buf1)
        return carry

    lax.fori_loop(0, nchunk // 2, pair_body, 0)
    pltpu.sync_copy(hist_v, out_hbm.at[wid])


def _make_hist(n):
    per_w = n // NWORKERS
    assert per_w * NWORKERS == n and per_w % (2 * CHUNK) == 0
    nchunk = per_w // CHUNK
    return pl.kernel(
        functools.partial(_hist_body, nchunk),
        out_type=jax.ShapeDtypeStruct((NWORKERS, SUBHIST), jnp.float32),
        mesh=plsc.VectorSubcoreMesh(
            core_axis_name="c", subcore_axis_name="s",
            num_cores=NCORES, num_subcores=NSUB,
        ),
        scratch_types=[
            pltpu.VMEM((CHUNK,), jnp.float32),
            pltpu.VMEM((CHUNK,), jnp.float32),
            pltpu.VMEM((SUBHIST,), jnp.float32),
            pltpu.SemaphoreType.DMA,
            pltpu.SemaphoreType.DMA,
        ],
        compiler_params=pltpu.CompilerParams(needs_layout_passes=False),
    )


def _otsu_body(hists_ref, out_ref):
    a = hists_ref[...]
    c = jnp.sum(a, axis=0, keepdims=True)
    jv = (lax.broadcasted_iota(jnp.int32, (1, SUBHIST), 1) >> 4).astype(
        jnp.float32)
    jj = lax.broadcasted_iota(jnp.int32, (SUBHIST, 1), 0) >> 4
    ii = lax.broadcasted_iota(jnp.int32, (1, NBINS), 1)
    k1 = (jj <= ii).astype(jnp.float32)

    w1 = lax.dot_general(
        c, k1, (((1,), (0,)), ((), ())),
        precision=lax.Precision.HIGHEST, preferred_element_type=jnp.float32,
    )
    cs1 = lax.dot_general(
        c * jv, k1, (((1,), (0,)), ((), ())),
        precision=lax.Precision.HIGHEST, preferred_element_type=jnp.float32,
    )

    n_tot = jnp.max(w1)
    s_tot = jnp.max(cs1)
    w2 = n_tot - w1
    valid = jnp.logical_and(w1 > 0.0, w2 > 0.0)
    var12 = w1 * w2 * (cs1 / w1 - (s_tot - cs1) / w2) ** 2
    var12 = jnp.where(valid, var12, -jnp.inf)
    m = jnp.max(var12)
    centers = lax.broadcasted_iota(jnp.int32, (1, NBINS), 1).astype(jnp.float32)
    cand = jnp.where(var12 == m, centers, jnp.float32(NBINS))
    out_ref[...] = jnp.reshape(jnp.min(cand), (1, 1))


_otsu = pl.pallas_call(
    _otsu_body,
    out_shape=jax.ShapeDtypeStruct((1, 1), jnp.float32),
)


@jax.jit
def kernel(x):
    hists = _make_hist(x.size)(x)
    thr = _otsu(hists)
    return thr[0, 0]


# speedup vs baseline: 1.0801x; 1.0063x over previous
import functools

import jax
import jax.numpy as jnp
import numpy as np
from jax import lax
from jax.experimental import pallas as pl
from jax.experimental.pallas import tpu as pltpu
from jax.experimental.pallas import tpu_sc as plsc

NBINS = 256
LANES = 16
NCORES = 2
NSUB = 16
NWORKERS = NCORES * NSUB
SUBHIST = NBINS * LANES
CHUNK = 32768
UNROLL = 16


def _hist_body(nchunk, x_hbm, out_hbm, buf0, buf1, hist_v, sem0, sem1):
    cid = lax.axis_index("c")
    sid = lax.axis_index("s")
    wid = sid * NCORES + cid
    per_w = nchunk * CHUNK
    base = wid * per_w

    zeros16 = jnp.zeros((LANES,), jnp.float32)
    ones16 = jnp.ones((LANES,), jnp.float32)
    lane = lax.iota(jnp.int32, LANES)
    lane_magic = lane.astype(jnp.float32) + jnp.float32(8388608.0)

    def zero_body(i, carry):
        for k in range(16):
            hist_v[pl.ds(i * (16 * LANES) + k * LANES, LANES)] = zeros16
        return carry

    lax.fori_loop(0, SUBHIST // (16 * LANES), zero_body, 0)

    def start(g, buf, sem):
        pltpu.async_copy(x_hbm.at[pl.ds(base + g * CHUNK, CHUNK)], buf, sem)

    def wait(buf, sem):
        pltpu.make_async_copy(x_hbm.at[pl.ds(base, CHUNK)], buf, sem).wait()

    def process(buf):
        def vec_body(i, c2):
            b0 = i * (LANES * UNROLL)
            vs = [buf[pl.ds(b0 + k * LANES, LANES)] for k in range(UNROLL)]
            fs = [v * jnp.float32(16.0) + lane_magic for v in vs]
            idxs = [plsc.bitcast(f, jnp.int32) & jnp.int32(0xFFF) for f in fs]
            for idx in idxs:
                plsc.addupdate_scatter(hist_v, [idx], ones16)
            return c2

        lax.fori_loop(0, CHUNK // (LANES * UNROLL), vec_body, 0)

    start(0, buf0, sem0)

    def pair_body(p, carry):
        g0 = 2 * p
        wait(buf0, sem0)
        start(g0 + 1, buf1, sem1)
        process(buf0)
        wait(buf1, sem1)

        @pl.when(g0 + 2 < nchunk)
        def _():
            start(g0 + 2, buf0, sem0)

        process(buf1)
        return carry

    lax.fori_loop(0, nchunk // 2, pair_body, 0)
    pltpu.sync_copy(hist_v, out_hbm.at[wid])


def _make_hist(n):
    per_w = n // NWORKERS
    assert per_w * NWORKERS == n and per_w % (2 * CHUNK) == 0
    nchunk = per_w // CHUNK
    return pl.kernel(
        functools.partial(_hist_body, nchunk),
        out_type=jax.ShapeDtypeStruct((NWORKERS, SUBHIST), jnp.float32),
        mesh=plsc.VectorSubcoreMesh(
            core_axis_name="c", subcore_axis_name="s",
            num_cores=NCORES, num_subcores=NSUB,
        ),
        scratch_types=[
            pltpu.VMEM((CHUNK,), jnp.float32),
            pltpu.VMEM((CHUNK,), jnp.float32),
            pltpu.VMEM((SUBHIST,), jnp.float32),
            pltpu.SemaphoreType.DMA,
            pltpu.SemaphoreType.DMA,
        ],
        compiler_params=pltpu.CompilerParams(
            needs_layout_passes=False, skip_device_barrier=True,
        ),
    )


def _otsu_body(hists_ref, out_ref):
    a = hists_ref[...]
    c = jnp.sum(a, axis=0, keepdims=True)
    jv = (lax.broadcasted_iota(jnp.int32, (1, SUBHIST), 1) >> 4).astype(
        jnp.float32)
    jj = lax.broadcasted_iota(jnp.int32, (SUBHIST, 1), 0) >> 4
    ii = lax.broadcasted_iota(jnp.int32, (1, NBINS), 1)
    k1 = (jj <= ii).astype(jnp.float32)

    w1 = lax.dot_general(
        c, k1, (((1,), (0,)), ((), ())),
        precision=lax.Precision.HIGHEST, preferred_element_type=jnp.float32,
    )
    cs1 = lax.dot_general(
        c * jv, k1, (((1,), (0,)), ((), ())),
        precision=lax.Precision.HIGHEST, preferred_element_type=jnp.float32,
    )

    n_tot = jnp.max(w1)
    s_tot = jnp.max(cs1)
    w2 = n_tot - w1
    valid = jnp.logical_and(w1 > 0.0, w2 > 0.0)
    var12 = w1 * w2 * (cs1 / w1 - (s_tot - cs1) / w2) ** 2
    var12 = jnp.where(valid, var12, -jnp.inf)
    m = jnp.max(var12)
    centers = lax.broadcasted_iota(jnp.int32, (1, NBINS), 1).astype(jnp.float32)
    cand = jnp.where(var12 == m, centers, jnp.float32(NBINS))
    out_ref[...] = jnp.reshape(jnp.min(cand), (1, 1))


_otsu = pl.pallas_call(
    _otsu_body,
    out_shape=jax.ShapeDtypeStruct((1, 1), jnp.float32),
)


@jax.jit
def kernel(x):
    hists = _make_hist(x.size)(x)
    thr = _otsu(hists)
    return thr[0, 0]
